# baseline (speedup 1.0000x reference)
import functools

import jax
import jax.numpy as jnp
from jax import lax
from jax.experimental import pallas as pl
from jax.experimental.pallas import tpu as pltpu
from jax.experimental.pallas import tpu_sc as plsc

_NC = 2
_NS = 16


def _sc_copy(table_hbm, out_hbm, buf, in_sems, out_sems, *, rows_per_w, chunk,
             nslots):
    wid = lax.axis_index("s") * _NC + lax.axis_index("c")
    base = wid * rows_per_w
    n = rows_per_w // chunk

    def in_copy(j, slot):
        return pltpu.make_async_copy(
            table_hbm.at[pl.ds(base + j * chunk, chunk), :], buf.at[slot],
            in_sems.at[slot])

    def out_copy(j, slot):
        return pltpu.make_async_copy(
            buf.at[slot], out_hbm.at[pl.ds(base + j * chunk, chunk), :],
            out_sems.at[slot])

    for j in range(min(nslots, n)):
        in_copy(j, j).start()
    for j in range(n):
        if j >= 1 and j + nslots - 1 < n:
            out_copy(j - 1, (j - 1) % nslots).wait()
            in_copy(j + nslots - 1, (j - 1) % nslots).start()
        in_copy(j, j % nslots).wait()
        out_copy(j, j % nslots).start()
    for j in range(max(0, n - nslots), n):
        out_copy(j, j % nslots).wait()


def kernel(inputs, table):
    seq_len = inputs.shape[-1]
    rows, dim = table.shape
    assert seq_len == rows
    nw = _NC * _NS
    rows_per_w = rows // nw
    chunk = 16
    nslots = 4
    mesh = plsc.VectorSubcoreMesh(core_axis_name="c", subcore_axis_name="s")
    f = functools.partial(_sc_copy, rows_per_w=rows_per_w, chunk=chunk,
                          nslots=nslots)
    return pl.kernel(
        f,
        mesh=mesh,
        out_type=jax.ShapeDtypeStruct((rows, dim), table.dtype),
        scratch_types=[
            pltpu.VMEM((nslots, chunk, dim), table.dtype),
            pltpu.SemaphoreType.DMA((nslots,)),
            pltpu.SemaphoreType.DMA((nslots,)),
        ],
    )(table)


# speedup vs baseline: 2.7198x; 2.7198x over previous
import functools

import jax
import jax.numpy as jnp
from jax.experimental import pallas as pl
from jax.experimental.pallas import tpu as pltpu


def _ring_copy(table_hbm, out_hbm, vmem, in_sems, out_sems, *, block, nslots):
    n = pl.num_programs(0)
    i = pl.program_id(0)

    def in_copy(j, slot):
        return pltpu.make_async_copy(
            table_hbm.at[pl.ds(j * block, block), :], vmem.at[slot],
            in_sems.at[slot])

    def out_copy(j, slot):
        return pltpu.make_async_copy(
            vmem.at[slot], out_hbm.at[pl.ds(j * block, block), :],
            out_sems.at[slot])

    @pl.when(i == 0)
    def _():
        for j in range(nslots):
            if j == 0:
                in_copy(0, 0).start()
            else:
                @pl.when(j < n)
                def _(j=j):
                    in_copy(j, j).start()

    @pl.when((i >= 1) & (i + nslots - 1 < n))
    def _():
        out_copy(i - 1, (i - 1) % nslots).wait()
        in_copy(i + nslots - 1, (i - 1) % nslots).start()

    in_copy(i, i % nslots).wait()
    out_copy(i, i % nslots).start()

    @pl.when(i == n - 1)
    def _():
        for j in range(nslots):
            @pl.when((i - j >= 0) & (i - j >= n - nslots))
            def _(j=j):
                out_copy(i - j, (i - j) % nslots).wait()


def kernel(inputs, table):
    seq_len = inputs.shape[-1]
    rows, dim = table.shape
    assert seq_len == rows
    block = 512
    nslots = 8
    n = rows // block
    return pl.pallas_call(
        functools.partial(_ring_copy, block=block, nslots=nslots),
        grid=(n,),
        in_specs=[pl.BlockSpec(memory_space=pl.ANY)],
        out_specs=pl.BlockSpec(memory_space=pl.ANY),
        out_shape=jax.ShapeDtypeStruct((rows, dim), table.dtype),
        scratch_shapes=[
            pltpu.VMEM((nslots, block, dim), table.dtype),
            pltpu.SemaphoreType.DMA((nslots,)),
            pltpu.SemaphoreType.DMA((nslots,)),
        ],
    )(table)
